# baseline (speedup 1.0000x reference)
import functools
import math

import jax
import jax.numpy as jnp
from jax import lax
from jax.experimental import pallas as pl
from jax.experimental.pallas import tpu as pltpu
from jax.experimental.pallas import tpu_sc as plsc

D_MODEL = 64
SCALE = math.sqrt(D_MODEL)

NUM_CORES = 2
NUM_SUBCORES = 16
NUM_WORKERS = NUM_CORES * NUM_SUBCORES

B_TOTAL = 4096 * 200
B_PER_W = B_TOTAL // NUM_WORKERS
CHUNK = 800
NCHUNK = B_PER_W // CHUNK
LANES = 16


def _sc_body(x_hbm, lut_hbm, out_hbm, idx_v, rows0, rows1, gsem0, gsem1, ssem):
    wid = lax.axis_index("s") * NUM_CORES + lax.axis_index("c")
    base = wid * B_PER_W

    pltpu.sync_copy(x_hbm.at[pl.ds(base, B_PER_W)], idx_v)

    bufs = (rows0, rows1)
    gsems = (gsem0, gsem1)

    def gather_start(g, buf, gsem):
        pltpu.async_copy(lut_hbm.at[idx_v.at[pl.ds(g * CHUNK, CHUNK)]], buf, gsem)

    def gather_wait(buf, gsem):
        pltpu.make_async_copy(lut_hbm.at[idx_v.at[pl.ds(0, CHUNK)]], buf, gsem).wait()

    def scale_rows(buf):
        def row_body(r, carry):
            for c in range(D_MODEL // LANES):
                sl = pl.ds(c * LANES, LANES)
                buf[r, sl] = buf[r, sl] * SCALE
            return carry

        lax.fori_loop(0, CHUNK, row_body, 0, unroll=2)

    gather_start(0, rows0, gsem0)

    def pair_body(t, carry):
        for b in range(2):
            g = 2 * t + b
            buf, gsem = bufs[b], gsems[b]
            nbuf, ngsem = bufs[1 - b], gsems[1 - b]
            gather_wait(buf, gsem)

            @pl.when(g + 1 < NCHUNK)
            def _():
                gather_start(g + 1, nbuf, ngsem)

            scale_rows(buf)
            pltpu.async_copy(buf, out_hbm.at[pl.ds(base + g * CHUNK, CHUNK)], ssem)
            pltpu.make_async_copy(buf, out_hbm.at[pl.ds(0, CHUNK)], ssem).wait()
        return carry

    lax.fori_loop(0, NCHUNK // 2, pair_body, 0)


@jax.jit
def _embed(x_flat, lut):
    mesh = plsc.VectorSubcoreMesh(core_axis_name="c", subcore_axis_name="s")
    k = pl.kernel(
        _sc_body,
        out_type=jax.ShapeDtypeStruct((B_TOTAL, D_MODEL), jnp.float32),
        mesh=mesh,
        compiler_params=pltpu.CompilerParams(use_tc_tiling_on_sc=False),
        scratch_types=[
            pltpu.VMEM((B_PER_W,), jnp.int32),
            pltpu.VMEM((CHUNK, D_MODEL), jnp.float32),
            pltpu.VMEM((CHUNK, D_MODEL), jnp.float32),
            pltpu.SemaphoreType.DMA,
            pltpu.SemaphoreType.DMA,
            pltpu.SemaphoreType.DMA,
        ],
    )
    return k(x_flat, lut)


def kernel(x, lut):
    out = _embed(x.reshape(-1), lut)
    return out.reshape(x.shape[0], x.shape[1], D_MODEL)
